# baseline (speedup 1.0000x reference)
import jax
import jax.numpy as jnp
from jax.experimental import pallas as pl

N, D, R, C = 4096, 128, 256, 16
BN = 256
RB = 32

NEG_INF = float("-inf")


def _rule_kernel(x_ref, lo_ref, up_ref, aw_ref, ro_ref, rw_ref, out_ref):
    x = x_ref[...]

    def body(i, carry):
        best, arg = carry
        sl = pl.ds(i * RB, RB)
        lo_c = lo_ref[sl, :]
        up_c = up_ref[sl, :]
        mask_c = aw_ref[sl, :] <= 0.0
        valid_c = jnp.logical_not(jnp.all(mask_c, axis=1))
        ro_c = ro_ref[i]

        ok = (x[:, None, :] > lo_c[None, :, :]) & (x[:, None, :] < up_c[None, :, :])
        fires = jnp.all(ok | mask_c[None, :, :], axis=2) & valid_c[None, :]
        score = jnp.where(fires, ro_c, NEG_INF)
        c_best = jnp.max(score, axis=1, keepdims=True)
        ii = jax.lax.broadcasted_iota(jnp.int32, score.shape, 1)
        cand = jnp.where(score == c_best, ii, RB)
        c_arg = jnp.min(cand, axis=1, keepdims=True) + i * RB
        upd = c_best > best
        best = jnp.where(upd, c_best, best)
        arg = jnp.where(upd, c_arg, arg)
        return best, arg

    init = (jnp.full((BN, 1), NEG_INF, jnp.float32), jnp.zeros((BN, 1), jnp.int32))
    best, arg = jax.lax.fori_loop(0, R // RB, body, init)

    covered = best > NEG_INF
    rr = jax.lax.broadcasted_iota(jnp.int32, (BN, R), 1)
    onehot = ((rr == arg) & covered).astype(jnp.float32)
    out_ref[...] = jnp.dot(onehot, rw_ref[...], preferred_element_type=jnp.float32,
                           precision=jax.lax.Precision.HIGHEST)


def kernel(x, cut_points, and_weights, rule_order, rule_weights):
    lower = cut_points[:, 0, :].T
    upper = cut_points[:, 1, :].T
    ro3 = rule_order.reshape(R // RB, 1, RB)
    grid = (N // BN,)
    return pl.pallas_call(
        _rule_kernel,
        grid=grid,
        in_specs=[
            pl.BlockSpec((BN, D), lambda i: (i, 0)),
            pl.BlockSpec((R, D), lambda i: (0, 0)),
            pl.BlockSpec((R, D), lambda i: (0, 0)),
            pl.BlockSpec((R, D), lambda i: (0, 0)),
            pl.BlockSpec((R // RB, 1, RB), lambda i: (0, 0, 0)),
            pl.BlockSpec((R, C), lambda i: (0, 0)),
        ],
        out_specs=pl.BlockSpec((BN, C), lambda i: (i, 0)),
        out_shape=jax.ShapeDtypeStruct((N, C), jnp.float32),
    )(x, lower, upper, and_weights, ro3, rule_weights)


# speedup vs baseline: 6.3374x; 6.3374x over previous
import jax
import jax.numpy as jnp
from jax import lax
from jax.experimental import pallas as pl
from jax.experimental.pallas import tpu as pltpu

N, D, R, C = 4096, 128, 256, 16
K = 8
BOUND = (-1.15, -0.67, -0.32, 0.0, 0.32, 0.67, 1.15)
BN = 512
NEG_INF = float("-inf")
HI = jax.lax.Precision.HIGHEST


def _rule_kernel(x_ref, lowT_ref, upT_ref, awT_ref, low_ref, up_ref, aw_ref,
                 ro_ref, rw_ref, out_ref, score_s, res_s):
    lowT = lowT_ref[...]
    upT = upT_ref[...]
    actT = awT_ref[...] > 0.0

    blo = (NEG_INF,) + BOUND
    bhi = BOUND + (float("inf"),)
    blocks = []
    for k in range(K):
        cf = (lowT >= bhi[k]) if bhi[k] != float("inf") else (lowT != lowT)
        if blo[k] != NEG_INF:
            cf = cf | (upT <= blo[k])
        blocks.append((actT & cf).astype(jnp.bfloat16))
    table = jnp.concatenate(blocks, axis=0)

    x = x_ref[...]
    b = (x >= BOUND[0]).astype(jnp.int32)
    for k in range(1, K - 1):
        b = b + (x >= BOUND[k]).astype(jnp.int32)
    oh = [(b == k).astype(jnp.bfloat16) for k in range(K)]
    onehot = jnp.concatenate(oh, axis=1)

    counts = jnp.dot(onehot, table, preferred_element_type=jnp.float32)
    valid = jnp.any(actT, axis=0)
    counts = counts + (1.0 - valid.astype(jnp.float32))[None, :]

    act = aw_ref[...] > 0.0
    lp = jnp.where(act, low_ref[...], -3.0e38)
    up = jnp.where(act, up_ref[...], 3.0e38)
    ro = ro_ref[...]
    rw = rw_ref[...]
    rr_iota = lax.broadcasted_iota(jnp.int32, (BN, R), 1)

    score_s[...] = jnp.where(counts == 0.0, jnp.broadcast_to(ro, (BN, R)),
                             NEG_INF)
    res_s[...] = jnp.zeros((BN, 1), jnp.float32)
    out_ref[...] = jnp.zeros((BN, C), jnp.float32)

    def w_cond(go):
        return go

    def w_body(go):
        score = score_s[...]
        resolved = res_s[...] > 0.0
        cm = jnp.max(score, axis=1, keepdims=True)
        nocand = cm == NEG_INF
        chosen = jnp.min(jnp.where(score == cm, rr_iota, R), axis=1,
                         keepdims=True)
        active = jnp.logical_not(resolved) & jnp.logical_not(nocand)
        ohs = ((rr_iota == chosen) & active).astype(jnp.float32)
        lr = jnp.dot(ohs, lp, preferred_element_type=jnp.float32, precision=HI)
        ur = jnp.dot(ohs, up, preferred_element_type=jnp.float32, precision=HI)
        fires = jnp.all((x > lr) & (x < ur), axis=1, keepdims=True) & active
        out_ref[...] += jnp.dot(ohs * fires.astype(jnp.float32), rw,
                                preferred_element_type=jnp.float32,
                                precision=HI)
        resolved = resolved | fires | nocand
        res_s[...] = resolved.astype(jnp.float32)
        kill = (rr_iota == chosen) & active & jnp.logical_not(fires)
        score_s[...] = jnp.where(kill, NEG_INF, score)
        return jnp.any(jnp.logical_not(resolved))

    lax.while_loop(w_cond, w_body, jnp.bool_(True))


def kernel(x, cut_points, and_weights, rule_order, rule_weights):
    lowT = cut_points[:, 0, :]
    upT = cut_points[:, 1, :]
    awT = and_weights.T
    low = lowT.T
    up = upT.T
    ro2 = rule_order.reshape(1, R)

    return pl.pallas_call(
        _rule_kernel,
        grid=(N // BN,),
        in_specs=[
            pl.BlockSpec((BN, D), lambda i: (i, 0)),
            pl.BlockSpec((D, R), lambda i: (0, 0)),
            pl.BlockSpec((D, R), lambda i: (0, 0)),
            pl.BlockSpec((D, R), lambda i: (0, 0)),
            pl.BlockSpec((R, D), lambda i: (0, 0)),
            pl.BlockSpec((R, D), lambda i: (0, 0)),
            pl.BlockSpec((R, D), lambda i: (0, 0)),
            pl.BlockSpec((1, R), lambda i: (0, 0)),
            pl.BlockSpec((R, C), lambda i: (0, 0)),
        ],
        out_specs=pl.BlockSpec((BN, C), lambda i: (i, 0)),
        out_shape=jax.ShapeDtypeStruct((N, C), jnp.float32),
        scratch_shapes=[
            pltpu.VMEM((BN, R), jnp.float32),
            pltpu.VMEM((BN, 1), jnp.float32),
        ],
    )(x, lowT, upT, awT, low, up, and_weights, ro2, rule_weights)


# speedup vs baseline: 7.1740x; 1.1320x over previous
import jax
import jax.numpy as jnp
from jax import lax
from jax.experimental import pallas as pl
from jax.experimental.pallas import tpu as pltpu

N, D, R, C = 4096, 128, 256, 16
K = 8
BOUND = (-1.15, -0.67, -0.32, 0.0, 0.32, 0.67, 1.15)
BN = 1024
NEG_INF = float("-inf")
HI = jax.lax.Precision.HIGHEST


def _rule_kernel(x_ref, lowT_ref, upT_ref, awT_ref, low_ref, up_ref, aw_ref,
                 ro_ref, rw_ref, out_ref, score_s, res_s):
    lowT = lowT_ref[...]
    upT = upT_ref[...]
    actT = awT_ref[...] > 0.0

    blo = (NEG_INF,) + BOUND
    bhi = BOUND + (float("inf"),)
    blocks = []
    for k in range(K):
        cf = (lowT >= bhi[k]) if bhi[k] != float("inf") else (lowT != lowT)
        if blo[k] != NEG_INF:
            cf = cf | (upT <= blo[k])
        blocks.append((actT & cf).astype(jnp.bfloat16))
    table = jnp.concatenate(blocks, axis=0)

    x = x_ref[...]
    b = (x >= BOUND[0]).astype(jnp.int32)
    for k in range(1, K - 1):
        b = b + (x >= BOUND[k]).astype(jnp.int32)
    oh = [(b == k).astype(jnp.bfloat16) for k in range(K)]
    onehot = jnp.concatenate(oh, axis=1)

    counts = jnp.dot(onehot, table, preferred_element_type=jnp.float32)
    valid = jnp.any(actT, axis=0)
    counts = counts + (1.0 - valid.astype(jnp.float32))[None, :]

    act = aw_ref[...] > 0.0
    lp = jnp.where(act, low_ref[...], -3.0e38)
    up = jnp.where(act, up_ref[...], 3.0e38)
    ro = ro_ref[...]
    rw = rw_ref[...]
    rr_iota = lax.broadcasted_iota(jnp.int32, (BN, R), 1)

    score_s[...] = jnp.where(counts == 0.0, jnp.broadcast_to(ro, (BN, R)),
                             NEG_INF)
    res_s[...] = jnp.zeros((BN, 1), jnp.float32)
    out_ref[...] = jnp.zeros((BN, C), jnp.float32)

    def w_cond(go):
        return go

    def w_body(go):
        score = score_s[...]
        resolved = res_s[...] > 0.0
        cm = jnp.max(score, axis=1, keepdims=True)
        nocand = cm == NEG_INF
        chosen = jnp.min(jnp.where(score == cm, rr_iota, R), axis=1,
                         keepdims=True)
        active = jnp.logical_not(resolved) & jnp.logical_not(nocand)
        ohs = ((rr_iota == chosen) & active).astype(jnp.float32)
        lr = jnp.dot(ohs, lp, preferred_element_type=jnp.float32, precision=HI)
        ur = jnp.dot(ohs, up, preferred_element_type=jnp.float32, precision=HI)
        fires = jnp.all((x > lr) & (x < ur), axis=1, keepdims=True) & active
        out_ref[...] += jnp.dot(ohs * fires.astype(jnp.float32), rw,
                                preferred_element_type=jnp.float32,
                                precision=HI)
        resolved = resolved | fires | nocand
        res_s[...] = resolved.astype(jnp.float32)
        kill = (rr_iota == chosen) & active & jnp.logical_not(fires)
        score_s[...] = jnp.where(kill, NEG_INF, score)
        return jnp.any(jnp.logical_not(resolved))

    lax.while_loop(w_cond, w_body, jnp.bool_(True))


def kernel(x, cut_points, and_weights, rule_order, rule_weights):
    lowT = cut_points[:, 0, :]
    upT = cut_points[:, 1, :]
    awT = and_weights.T
    low = lowT.T
    up = upT.T
    ro2 = rule_order.reshape(1, R)

    return pl.pallas_call(
        _rule_kernel,
        grid=(N // BN,),
        in_specs=[
            pl.BlockSpec((BN, D), lambda i: (i, 0)),
            pl.BlockSpec((D, R), lambda i: (0, 0)),
            pl.BlockSpec((D, R), lambda i: (0, 0)),
            pl.BlockSpec((D, R), lambda i: (0, 0)),
            pl.BlockSpec((R, D), lambda i: (0, 0)),
            pl.BlockSpec((R, D), lambda i: (0, 0)),
            pl.BlockSpec((R, D), lambda i: (0, 0)),
            pl.BlockSpec((1, R), lambda i: (0, 0)),
            pl.BlockSpec((R, C), lambda i: (0, 0)),
        ],
        out_specs=pl.BlockSpec((BN, C), lambda i: (i, 0)),
        out_shape=jax.ShapeDtypeStruct((N, C), jnp.float32),
        scratch_shapes=[
            pltpu.VMEM((BN, R), jnp.float32),
            pltpu.VMEM((BN, 1), jnp.float32),
        ],
    )(x, lowT, upT, awT, low, up, and_weights, ro2, rule_weights)


# speedup vs baseline: 7.2414x; 1.0094x over previous
import jax
import jax.numpy as jnp
from jax import lax
from jax.experimental import pallas as pl
from jax.experimental.pallas import tpu as pltpu

N, D, R, C = 4096, 128, 256, 16
K = 8
BOUND = (-1.15, -0.67, -0.32, 0.0, 0.32, 0.67, 1.15)
BN = 2048
NEG_INF = float("-inf")
HI = jax.lax.Precision.HIGHEST


def _rule_kernel(x_ref, lowT_ref, upT_ref, awT_ref, low_ref, up_ref, aw_ref,
                 ro_ref, rw_ref, out_ref, score_s, res_s):
    lowT = lowT_ref[...]
    upT = upT_ref[...]
    actT = awT_ref[...] > 0.0

    blo = (NEG_INF,) + BOUND
    bhi = BOUND + (float("inf"),)
    blocks = []
    for k in range(K):
        cf = (lowT >= bhi[k]) if bhi[k] != float("inf") else (lowT != lowT)
        if blo[k] != NEG_INF:
            cf = cf | (upT <= blo[k])
        blocks.append((actT & cf).astype(jnp.bfloat16))
    table = jnp.concatenate(blocks, axis=0)

    x = x_ref[...]
    b = (x >= BOUND[0]).astype(jnp.int32)
    for k in range(1, K - 1):
        b = b + (x >= BOUND[k]).astype(jnp.int32)
    oh = [(b == k).astype(jnp.bfloat16) for k in range(K)]
    onehot = jnp.concatenate(oh, axis=1)

    counts = jnp.dot(onehot, table, preferred_element_type=jnp.float32)
    valid = jnp.any(actT, axis=0)
    counts = counts + (1.0 - valid.astype(jnp.float32))[None, :]

    act = aw_ref[...] > 0.0
    lp = jnp.where(act, low_ref[...], -3.0e38)
    up = jnp.where(act, up_ref[...], 3.0e38)
    ro = ro_ref[...]
    rw = rw_ref[...]
    rr_iota = lax.broadcasted_iota(jnp.int32, (BN, R), 1)

    score_s[...] = jnp.where(counts == 0.0, jnp.broadcast_to(ro, (BN, R)),
                             NEG_INF)
    res_s[...] = jnp.zeros((BN, 1), jnp.float32)
    out_ref[...] = jnp.zeros((BN, C), jnp.float32)

    def w_cond(go):
        return go

    def w_body(go):
        score = score_s[...]
        resolved = res_s[...] > 0.0
        cm = jnp.max(score, axis=1, keepdims=True)
        nocand = cm == NEG_INF
        chosen = jnp.min(jnp.where(score == cm, rr_iota, R), axis=1,
                         keepdims=True)
        active = jnp.logical_not(resolved) & jnp.logical_not(nocand)
        ohs = ((rr_iota == chosen) & active).astype(jnp.float32)
        lr = jnp.dot(ohs, lp, preferred_element_type=jnp.float32, precision=HI)
        ur = jnp.dot(ohs, up, preferred_element_type=jnp.float32, precision=HI)
        fires = jnp.all((x > lr) & (x < ur), axis=1, keepdims=True) & active
        out_ref[...] += jnp.dot(ohs * fires.astype(jnp.float32), rw,
                                preferred_element_type=jnp.float32,
                                precision=HI)
        resolved = resolved | fires | nocand
        res_s[...] = resolved.astype(jnp.float32)
        kill = (rr_iota == chosen) & active & jnp.logical_not(fires)
        score_s[...] = jnp.where(kill, NEG_INF, score)
        return jnp.any(jnp.logical_not(resolved))

    lax.while_loop(w_cond, w_body, jnp.bool_(True))


def kernel(x, cut_points, and_weights, rule_order, rule_weights):
    lowT = cut_points[:, 0, :]
    upT = cut_points[:, 1, :]
    awT = and_weights.T
    low = lowT.T
    up = upT.T
    ro2 = rule_order.reshape(1, R)

    return pl.pallas_call(
        _rule_kernel,
        grid=(N // BN,),
        in_specs=[
            pl.BlockSpec((BN, D), lambda i: (i, 0)),
            pl.BlockSpec((D, R), lambda i: (0, 0)),
            pl.BlockSpec((D, R), lambda i: (0, 0)),
            pl.BlockSpec((D, R), lambda i: (0, 0)),
            pl.BlockSpec((R, D), lambda i: (0, 0)),
            pl.BlockSpec((R, D), lambda i: (0, 0)),
            pl.BlockSpec((R, D), lambda i: (0, 0)),
            pl.BlockSpec((1, R), lambda i: (0, 0)),
            pl.BlockSpec((R, C), lambda i: (0, 0)),
        ],
        out_specs=pl.BlockSpec((BN, C), lambda i: (i, 0)),
        out_shape=jax.ShapeDtypeStruct((N, C), jnp.float32),
        scratch_shapes=[
            pltpu.VMEM((BN, R), jnp.float32),
            pltpu.VMEM((BN, 1), jnp.float32),
        ],
    )(x, lowT, upT, awT, low, up, and_weights, ro2, rule_weights)
